# baseline (speedup 1.0000x reference)
import functools

import jax
import jax.numpy as jnp
from jax import lax
from jax.experimental import pallas as pl
from jax.experimental.pallas import tpu as pltpu
from jax.experimental.pallas import tpu_sc as plsc

N = 10000
E = 320000
D = 128
OUT = 3
G = 64

NC, NS, L = 2, 16, 16
NW = NC * NS

NP = 10240
EW = 10240
EP = NW * EW
CH = 128
NCHUNK_DEG = EW // CH
CHE = 80
NCHUNK = EW // CHE
RPS = NP // NS
BG = 128
BPW = NP // NW
CHP = 80

_mesh = plsc.VectorSubcoreMesh(core_axis_name="c", subcore_axis_name="s")
_f32 = jnp.float32


@functools.partial(
    pl.kernel,
    out_type=[jax.ShapeDtypeStruct((NW, NP), _f32),
              jax.ShapeDtypeStruct((NW, BG), _f32)],
    mesh=_mesh,
    scratch_types=[pltpu.VMEM((NP,), _f32),
                   pltpu.VMEM((EW,), jnp.int32),
                   pltpu.VMEM((BPW,), jnp.int32),
                   pltpu.VMEM((BG,), _f32)],
    compiler_params=pltpu.CompilerParams(needs_layout_passes=False),
)
def _deg(dst_hbm, batch_hbm, zeros1_hbm, ecnt_hbm, bcnt_hbm,
         cnt_v, idx_v, bidx_v, bcnt_v):
    c = lax.axis_index("c")
    s = lax.axis_index("s")
    wid = s * NC + c
    pltpu.sync_copy(zeros1_hbm, cnt_v)
    pltpu.sync_copy(zeros1_hbm.at[pl.ds(0, BG)], bcnt_v)
    pltpu.sync_copy(dst_hbm.at[wid], idx_v)
    ones = jnp.full((L,), 1.0, _f32)

    def chunk(i, carry):
        for k in range(CH // L):
            plsc.addupdate_scatter(
                cnt_v, [idx_v[pl.ds(i * CH + k * L, L)]], ones)
        return carry

    lax.fori_loop(0, NCHUNK_DEG, chunk, 0)
    pltpu.sync_copy(batch_hbm.at[pl.ds(wid * BPW, BPW)], bidx_v)
    for k in range(BPW // L):
        plsc.addupdate_scatter(bcnt_v, [bidx_v[pl.ds(k * L, L)]], ones)
    pltpu.sync_copy(cnt_v, ecnt_hbm.at[wid])
    pltpu.sync_copy(bcnt_v, bcnt_hbm.at[wid])


@functools.partial(
    pl.kernel,
    out_type=jax.ShapeDtypeStruct((NC, NP, D), _f32),
    mesh=_mesh,
    scratch_types=[pltpu.VMEM((EW,), jnp.int32),
                   pltpu.VMEM((NCHUNK, CHE), jnp.int32),
                   pltpu.VMEM((CHE, D), _f32),
                   pltpu.VMEM((CHE, D), _f32),
                   pltpu.VMEM_SHARED((NP, D), _f32),
                   pltpu.SemaphoreType.DMA,
                   pltpu.SemaphoreType.DMA,
                   pltpu.SemaphoreType.DMA,
                   pltpu.SemaphoreType.DMA],
)
def _edge_scatter(u_hbm, src_hbm, dst_hbm, zeros2_hbm, out_hbm,
                  si_v, di_v, rows0_v, rows1_v, acc_sh, g0, g1, s0, s1):
    c = lax.axis_index("c")
    s = lax.axis_index("s")
    wid = s * NC + c
    rs = s * RPS
    pltpu.sync_copy(src_hbm.at[wid], si_v)
    pltpu.sync_copy(dst_hbm.at[wid], di_v)
    pltpu.sync_copy(zeros2_hbm.at[pl.ds(rs, RPS)], acc_sh.at[pl.ds(rs, RPS)])
    plsc.subcore_barrier()

    def sidx(c):
        return si_v.at[pl.ds(c * CHE, CHE)]

    def gwait(rows_v, sem):
        pltpu.make_async_copy(u_hbm.at[sidx(0)], rows_v, sem).wait()

    def swait(rows_v, sem):
        pltpu.make_async_copy(rows_v, acc_sh.at[di_v.at[0]], sem).wait()

    pltpu.async_copy(u_hbm.at[sidx(0)], rows0_v, g0)
    pltpu.async_copy(u_hbm.at[sidx(1)], rows1_v, g1)

    def pair(i, carry):
        c0 = 2 * i
        gwait(rows0_v, g0)
        pltpu.async_copy(rows0_v, acc_sh.at[di_v.at[c0]], s0, add=True)
        gwait(rows1_v, g1)
        pltpu.async_copy(rows1_v, acc_sh.at[di_v.at[c0 + 1]], s1, add=True)

        @pl.when(i < NCHUNK // 2 - 1)
        def _():
            swait(rows0_v, s0)
            pltpu.async_copy(u_hbm.at[sidx(c0 + 2)], rows0_v, g0)
            swait(rows1_v, s1)
            pltpu.async_copy(u_hbm.at[sidx(c0 + 3)], rows1_v, g1)

        return carry

    lax.fori_loop(0, NCHUNK // 2, pair, 0)
    swait(rows0_v, s0)
    swait(rows1_v, s1)
    plsc.subcore_barrier()
    pltpu.sync_copy(acc_sh.at[pl.ds(rs, RPS)], out_hbm.at[c, pl.ds(rs, RPS)])


@functools.partial(
    pl.kernel,
    out_type=jax.ShapeDtypeStruct((NC, BG, D), _f32),
    mesh=_mesh,
    scratch_types=[pltpu.VMEM((CHP,), jnp.int32),
                   pltpu.VMEM((CHP, D), _f32),
                   pltpu.VMEM_SHARED((BG, D), _f32)],
)
def _pool(h_hbm, batch_hbm, zeros2_hbm, out_hbm, bidx_v, rows_v, acc_sh):
    c = lax.axis_index("c")
    s = lax.axis_index("s")
    wid = s * NC + c
    rs = s * (BG // NS)
    pltpu.sync_copy(zeros2_hbm.at[pl.ds(rs, BG // NS)],
                    acc_sh.at[pl.ds(rs, BG // NS)])
    plsc.subcore_barrier()
    base = wid * BPW
    for j in range(BPW // CHP):
        pltpu.sync_copy(batch_hbm.at[pl.ds(base + j * CHP, CHP)], bidx_v)
        pltpu.sync_copy(h_hbm.at[pl.ds(base + j * CHP, CHP)], rows_v)
        pltpu.sync_copy(rows_v, acc_sh.at[bidx_v], add=True)
    plsc.subcore_barrier()
    pltpu.sync_copy(acc_sh.at[pl.ds(rs, BG // NS)],
                    out_hbm.at[c, pl.ds(rs, BG // NS)])


R = 1024
NBLK = NP // R


def _prep1_body(x_ref, cnt_ref, w_ref, u_ref, dinv_ref):
    i = pl.program_id(0)
    tot = jnp.sum(cnt_ref[...], axis=1, keepdims=True)
    rows = i * R + lax.broadcasted_iota(jnp.int32, (R, 1), 0)
    dinv = jnp.where(rows < N, lax.rsqrt(tot + 1.0), 0.0)
    u_ref[...] = jnp.dot(x_ref[...], w_ref[...],
                         preferred_element_type=_f32) * dinv
    dinv_ref[...] = dinv


def _prep1(x_p, ecntT, W1):
    return pl.pallas_call(
        _prep1_body,
        grid=(NBLK,),
        in_specs=[pl.BlockSpec((R, D), lambda i: (i, 0)),
                  pl.BlockSpec((R, NW), lambda i: (i, 0)),
                  pl.BlockSpec((D, D), lambda i: (0, 0))],
        out_specs=[pl.BlockSpec((R, D), lambda i: (i, 0)),
                   pl.BlockSpec((R, 1), lambda i: (i, 0))],
        out_shape=[jax.ShapeDtypeStruct((NP, D), _f32),
                   jax.ShapeDtypeStruct((NP, 1), _f32)],
    )(x_p, ecntT, W1)


def _mid_body(y_ref, u_ref, dinv_ref, b_ref, w_ref, o_ref):
    ys = jnp.sum(y_ref[...], axis=0)
    dinv = dinv_ref[...]
    h = jnp.maximum((ys + u_ref[...]) * dinv + b_ref[...], 0.0)
    o_ref[...] = jnp.dot(h, w_ref[...], preferred_element_type=_f32) * dinv


def _mid(y, u, dinv, b, W):
    return pl.pallas_call(
        _mid_body,
        grid=(NBLK,),
        in_specs=[pl.BlockSpec((NC, R, D), lambda i: (0, i, 0)),
                  pl.BlockSpec((R, D), lambda i: (i, 0)),
                  pl.BlockSpec((R, 1), lambda i: (i, 0)),
                  pl.BlockSpec((1, D), lambda i: (0, 0)),
                  pl.BlockSpec((D, D), lambda i: (0, 0))],
        out_specs=pl.BlockSpec((R, D), lambda i: (i, 0)),
        out_shape=jax.ShapeDtypeStruct((NP, D), _f32),
    )(y, u, dinv, b, W)


def _last_body(y_ref, u_ref, dinv_ref, b_ref, o_ref):
    ys = jnp.sum(y_ref[...], axis=0)
    o_ref[...] = jnp.maximum((ys + u_ref[...]) * dinv_ref[...] + b_ref[...],
                             0.0)


def _last(y, u, dinv, b):
    return pl.pallas_call(
        _last_body,
        grid=(NBLK,),
        in_specs=[pl.BlockSpec((NC, R, D), lambda i: (0, i, 0)),
                  pl.BlockSpec((R, D), lambda i: (i, 0)),
                  pl.BlockSpec((R, 1), lambda i: (i, 0)),
                  pl.BlockSpec((1, D), lambda i: (0, 0))],
        out_specs=pl.BlockSpec((R, D), lambda i: (i, 0)),
        out_shape=jax.ShapeDtypeStruct((NP, D), _f32),
    )(y, u, dinv, b)


def _final_body(ps_ref, bcnt_ref, wc_ref, bc_ref, o_ref):
    sums = jnp.sum(ps_ref[...], axis=0)[:G]
    cnt = jnp.sum(bcnt_ref[...], axis=1, keepdims=True)[:G]
    pooled = sums / jnp.maximum(cnt, 1.0)
    o_ref[...] = jnp.dot(pooled, wc_ref[...],
                         preferred_element_type=_f32) + bc_ref[...]


def _final(ps, bcntT, Wc, bc):
    return pl.pallas_call(
        _final_body,
        out_shape=jax.ShapeDtypeStruct((G, OUT), _f32),
    )(ps, bcntT, Wc, bc)


def kernel(x, edge_index, batch, W1, b1, W2, b2, W3, b3, Wc, bc):
    src = edge_index[0]
    dst = edge_index[1]
    pad_idx = N + (jnp.arange(EP - E, dtype=jnp.int32) % (NP - N))
    src_p = jnp.concatenate([src, pad_idx])
    dst_p = jnp.concatenate([dst, pad_idx])
    src2 = src_p.reshape(NW, EW)
    dst2 = dst_p.reshape(NW, EW)
    dst3 = dst_p.reshape(NW, NCHUNK, CHE)
    batch_p = jnp.concatenate(
        [batch, G + (jnp.arange(NP - N, dtype=jnp.int32) % G)])
    x_p = jnp.zeros((NP, D), _f32).at[:N].set(x)
    zeros1 = jnp.zeros((NP,), _f32)
    zeros2 = jnp.zeros((NP, D), _f32)

    ecnt, bcnt = _deg(dst2, batch_p, zeros1)
    u1, dinv = _prep1(x_p, ecnt.T, W1)
    y1 = _edge_scatter(u1, src2, dst3, zeros2)
    u2 = _mid(y1, u1, dinv, b1.reshape(1, D), W2)
    y2 = _edge_scatter(u2, src2, dst3, zeros2)
    u3 = _mid(y2, u2, dinv, b2.reshape(1, D), W3)
    y3 = _edge_scatter(u3, src2, dst3, zeros2)
    h4 = _last(y3, u3, dinv, b3.reshape(1, D))
    ps = _pool(h4, batch_p, zeros2)
    return _final(ps, bcnt.T, Wc, bc.reshape(1, OUT))


# speedup vs baseline: 1.2245x; 1.2245x over previous
import functools

import jax
import jax.numpy as jnp
from jax import lax
from jax.experimental import pallas as pl
from jax.experimental.pallas import tpu as pltpu
from jax.experimental.pallas import tpu_sc as plsc

N = 10000
E = 320000
D = 128
OUT = 3
G = 64

NC, NS, L = 2, 16, 16
NW = NC * NS

NP = 10240
EW = 10240
EP = NW * EW
CH = 128
NCHUNK_DEG = EW // CH
CHE = 80
NCHUNK = EW // CHE
RPS = NP // NS
BG = 128
BPW = NP // NW
CHP = 80

_mesh = plsc.VectorSubcoreMesh(core_axis_name="c", subcore_axis_name="s")
_f32 = jnp.float32


@functools.partial(
    pl.kernel,
    out_type=[jax.ShapeDtypeStruct((NW, NP), _f32),
              jax.ShapeDtypeStruct((NW, BG), _f32)],
    mesh=_mesh,
    scratch_types=[pltpu.VMEM((NP,), _f32),
                   pltpu.VMEM((EW,), jnp.int32),
                   pltpu.VMEM((BPW,), jnp.int32),
                   pltpu.VMEM((BG,), _f32)],
    compiler_params=pltpu.CompilerParams(needs_layout_passes=False),
)
def _deg(dst_hbm, batch_hbm, zeros1_hbm, ecnt_hbm, bcnt_hbm,
         cnt_v, idx_v, bidx_v, bcnt_v):
    c = lax.axis_index("c")
    s = lax.axis_index("s")
    wid = s * NC + c
    pltpu.sync_copy(zeros1_hbm, cnt_v)
    pltpu.sync_copy(zeros1_hbm.at[pl.ds(0, BG)], bcnt_v)
    pltpu.sync_copy(dst_hbm.at[wid], idx_v)
    ones = jnp.full((L,), 1.0, _f32)

    def chunk(i, carry):
        for k in range(CH // L):
            plsc.addupdate_scatter(
                cnt_v, [idx_v[pl.ds(i * CH + k * L, L)]], ones)
        return carry

    lax.fori_loop(0, NCHUNK_DEG, chunk, 0)
    pltpu.sync_copy(batch_hbm.at[pl.ds(wid * BPW, BPW)], bidx_v)
    for k in range(BPW // L):
        plsc.addupdate_scatter(bcnt_v, [bidx_v[pl.ds(k * L, L)]], ones)
    pltpu.sync_copy(cnt_v, ecnt_hbm.at[wid])
    pltpu.sync_copy(bcnt_v, bcnt_hbm.at[wid])


@functools.partial(
    pl.kernel,
    out_type=jax.ShapeDtypeStruct((NC, NP, D), _f32),
    mesh=_mesh,
    scratch_types=[pltpu.VMEM((EW,), jnp.int32),
                   pltpu.VMEM((NCHUNK, CHE), jnp.int32),
                   pltpu.VMEM((CHE, D), _f32),
                   pltpu.VMEM((CHE, D), _f32),
                   pltpu.VMEM_SHARED((NP, D), _f32),
                   pltpu.SemaphoreType.DMA,
                   pltpu.SemaphoreType.DMA,
                   pltpu.SemaphoreType.DMA,
                   pltpu.SemaphoreType.DMA],
)
def _edge_scatter(u_hbm, src_hbm, dst_hbm, zeros2_hbm, out_hbm,
                  si_v, di_v, rows0_v, rows1_v, acc_sh, g0, g1, s0, s1):
    c = lax.axis_index("c")
    s = lax.axis_index("s")
    wid = s * NC + c
    rs = s * RPS
    pltpu.sync_copy(src_hbm.at[wid], si_v)
    pltpu.sync_copy(dst_hbm.at[wid], di_v)
    pltpu.sync_copy(zeros2_hbm.at[pl.ds(rs, RPS)], acc_sh.at[pl.ds(rs, RPS)])
    plsc.subcore_barrier()

    def sidx(c):
        return si_v.at[pl.ds(c * CHE, CHE)]

    pltpu.async_copy(u_hbm.at[sidx(0)], rows0_v, g0)

    def pair(i, carry):
        c0 = 2 * i
        pltpu.async_copy(u_hbm.at[sidx(c0 + 1)], rows1_v, g1)
        pltpu.make_async_copy(u_hbm.at[sidx(c0)], rows0_v, g0).wait()
        pltpu.sync_copy(rows0_v, acc_sh.at[di_v.at[c0]], add=True)

        @pl.when(i < NCHUNK // 2 - 1)
        def _():
            pltpu.async_copy(u_hbm.at[sidx(c0 + 2)], rows0_v, g0)

        pltpu.make_async_copy(u_hbm.at[sidx(c0 + 1)], rows1_v, g1).wait()
        pltpu.sync_copy(rows1_v, acc_sh.at[di_v.at[c0 + 1]], add=True)
        return carry

    lax.fori_loop(0, NCHUNK // 2, pair, 0)
    plsc.subcore_barrier()
    pltpu.sync_copy(acc_sh.at[pl.ds(rs, RPS)], out_hbm.at[c, pl.ds(rs, RPS)])


@functools.partial(
    pl.kernel,
    out_type=jax.ShapeDtypeStruct((NC, BG, D), _f32),
    mesh=_mesh,
    scratch_types=[pltpu.VMEM((CHP,), jnp.int32),
                   pltpu.VMEM((CHP, D), _f32),
                   pltpu.VMEM_SHARED((BG, D), _f32)],
)
def _pool(h_hbm, batch_hbm, zeros2_hbm, out_hbm, bidx_v, rows_v, acc_sh):
    c = lax.axis_index("c")
    s = lax.axis_index("s")
    wid = s * NC + c
    rs = s * (BG // NS)
    pltpu.sync_copy(zeros2_hbm.at[pl.ds(rs, BG // NS)],
                    acc_sh.at[pl.ds(rs, BG // NS)])
    plsc.subcore_barrier()
    base = wid * BPW
    for j in range(BPW // CHP):
        pltpu.sync_copy(batch_hbm.at[pl.ds(base + j * CHP, CHP)], bidx_v)
        pltpu.sync_copy(h_hbm.at[pl.ds(base + j * CHP, CHP)], rows_v)
        pltpu.sync_copy(rows_v, acc_sh.at[bidx_v], add=True)
    plsc.subcore_barrier()
    pltpu.sync_copy(acc_sh.at[pl.ds(rs, BG // NS)],
                    out_hbm.at[c, pl.ds(rs, BG // NS)])


R = 1024
NBLK = NP // R


def _prep1_body(x_ref, cnt_ref, w_ref, u_ref, dinv_ref):
    i = pl.program_id(0)
    tot = jnp.sum(cnt_ref[...], axis=1, keepdims=True)
    rows = i * R + lax.broadcasted_iota(jnp.int32, (R, 1), 0)
    dinv = jnp.where(rows < N, lax.rsqrt(tot + 1.0), 0.0)
    u_ref[...] = jnp.dot(x_ref[...], w_ref[...],
                         preferred_element_type=_f32) * dinv
    dinv_ref[...] = dinv


def _prep1(x_p, ecntT, W1):
    return pl.pallas_call(
        _prep1_body,
        grid=(NBLK,),
        in_specs=[pl.BlockSpec((R, D), lambda i: (i, 0)),
                  pl.BlockSpec((R, NW), lambda i: (i, 0)),
                  pl.BlockSpec((D, D), lambda i: (0, 0))],
        out_specs=[pl.BlockSpec((R, D), lambda i: (i, 0)),
                   pl.BlockSpec((R, 1), lambda i: (i, 0))],
        out_shape=[jax.ShapeDtypeStruct((NP, D), _f32),
                   jax.ShapeDtypeStruct((NP, 1), _f32)],
    )(x_p, ecntT, W1)


def _mid_body(y_ref, u_ref, dinv_ref, b_ref, w_ref, o_ref):
    ys = jnp.sum(y_ref[...], axis=0)
    dinv = dinv_ref[...]
    h = jnp.maximum((ys + u_ref[...]) * dinv + b_ref[...], 0.0)
    o_ref[...] = jnp.dot(h, w_ref[...], preferred_element_type=_f32) * dinv


def _mid(y, u, dinv, b, W):
    return pl.pallas_call(
        _mid_body,
        grid=(NBLK,),
        in_specs=[pl.BlockSpec((NC, R, D), lambda i: (0, i, 0)),
                  pl.BlockSpec((R, D), lambda i: (i, 0)),
                  pl.BlockSpec((R, 1), lambda i: (i, 0)),
                  pl.BlockSpec((1, D), lambda i: (0, 0)),
                  pl.BlockSpec((D, D), lambda i: (0, 0))],
        out_specs=pl.BlockSpec((R, D), lambda i: (i, 0)),
        out_shape=jax.ShapeDtypeStruct((NP, D), _f32),
    )(y, u, dinv, b, W)


def _last_body(y_ref, u_ref, dinv_ref, b_ref, o_ref):
    ys = jnp.sum(y_ref[...], axis=0)
    o_ref[...] = jnp.maximum((ys + u_ref[...]) * dinv_ref[...] + b_ref[...],
                             0.0)


def _last(y, u, dinv, b):
    return pl.pallas_call(
        _last_body,
        grid=(NBLK,),
        in_specs=[pl.BlockSpec((NC, R, D), lambda i: (0, i, 0)),
                  pl.BlockSpec((R, D), lambda i: (i, 0)),
                  pl.BlockSpec((R, 1), lambda i: (i, 0)),
                  pl.BlockSpec((1, D), lambda i: (0, 0))],
        out_specs=pl.BlockSpec((R, D), lambda i: (i, 0)),
        out_shape=jax.ShapeDtypeStruct((NP, D), _f32),
    )(y, u, dinv, b)


def _final_body(ps_ref, bcnt_ref, wc_ref, bc_ref, o_ref):
    sums = jnp.sum(ps_ref[...], axis=0)[:G]
    cnt = jnp.sum(bcnt_ref[...], axis=1, keepdims=True)[:G]
    pooled = sums / jnp.maximum(cnt, 1.0)
    o_ref[...] = jnp.dot(pooled, wc_ref[...],
                         preferred_element_type=_f32) + bc_ref[...]


def _final(ps, bcntT, Wc, bc):
    return pl.pallas_call(
        _final_body,
        out_shape=jax.ShapeDtypeStruct((G, OUT), _f32),
    )(ps, bcntT, Wc, bc)


def kernel(x, edge_index, batch, W1, b1, W2, b2, W3, b3, Wc, bc):
    src = edge_index[0]
    dst = edge_index[1]
    pad_idx = N + (jnp.arange(EP - E, dtype=jnp.int32) % (NP - N))
    src_p = jnp.concatenate([src, pad_idx])
    dst_p = jnp.concatenate([dst, pad_idx])
    src2 = src_p.reshape(NW, EW)
    dst2 = dst_p.reshape(NW, EW)
    dst3 = dst_p.reshape(NW, NCHUNK, CHE)
    batch_p = jnp.concatenate(
        [batch, G + (jnp.arange(NP - N, dtype=jnp.int32) % G)])
    x_p = jnp.zeros((NP, D), _f32).at[:N].set(x)
    zeros1 = jnp.zeros((NP,), _f32)
    zeros2 = jnp.zeros((NP, D), _f32)

    ecnt, bcnt = _deg(dst2, batch_p, zeros1)
    u1, dinv = _prep1(x_p, ecnt.T, W1)
    y1 = _edge_scatter(u1, src2, dst3, zeros2)
    u2 = _mid(y1, u1, dinv, b1.reshape(1, D), W2)
    y2 = _edge_scatter(u2, src2, dst3, zeros2)
    u3 = _mid(y2, u2, dinv, b2.reshape(1, D), W3)
    y3 = _edge_scatter(u3, src2, dst3, zeros2)
    h4 = _last(y3, u3, dinv, b3.reshape(1, D))
    ps = _pool(h4, batch_p, zeros2)
    return _final(ps, bcnt.T, Wc, bc.reshape(1, OUT))


# speedup vs baseline: 1.2582x; 1.0275x over previous
import functools

import jax
import jax.numpy as jnp
from jax import lax
from jax.experimental import pallas as pl
from jax.experimental.pallas import tpu as pltpu
from jax.experimental.pallas import tpu_sc as plsc

N = 10000
E = 320000
D = 128
OUT = 3
G = 64

NC, NS, L = 2, 16, 16
NW = NC * NS

NP = 10240
EW = 10240
EP = NW * EW
CH = 128
NCHUNK_DEG = EW // CH
CHE = 80
NCHUNK = EW // CHE
RPS = NP // NS
BG = 128
BPW = NP // NW
CHP = 80

_mesh = plsc.VectorSubcoreMesh(core_axis_name="c", subcore_axis_name="s")
_f32 = jnp.float32


@functools.partial(
    pl.kernel,
    out_type=jax.ShapeDtypeStruct((NW, NP), _f32),
    mesh=_mesh,
    scratch_types=[pltpu.VMEM((NP,), _f32),
                   pltpu.VMEM((EW,), jnp.int32)],
    compiler_params=pltpu.CompilerParams(needs_layout_passes=False),
)
def _deg(dst_hbm, zeros1_hbm, ecnt_hbm, cnt_v, idx_v):
    c = lax.axis_index("c")
    s = lax.axis_index("s")
    wid = s * NC + c
    pltpu.sync_copy(zeros1_hbm, cnt_v)
    pltpu.sync_copy(dst_hbm.at[wid], idx_v)
    ones = jnp.full((L,), 1.0, _f32)

    def chunk(i, carry):
        for k in range(CH // L):
            plsc.addupdate_scatter(
                cnt_v, [idx_v[pl.ds(i * CH + k * L, L)]], ones)
        return carry

    lax.fori_loop(0, NCHUNK_DEG, chunk, 0)
    pltpu.sync_copy(cnt_v, ecnt_hbm.at[wid])


@functools.partial(
    pl.kernel,
    out_type=jax.ShapeDtypeStruct((NC, NP, D), _f32),
    mesh=_mesh,
    scratch_types=[pltpu.VMEM((EW,), jnp.int32),
                   pltpu.VMEM((NCHUNK, CHE), jnp.int32),
                   pltpu.VMEM((CHE, D), _f32),
                   pltpu.VMEM((CHE, D), _f32),
                   pltpu.VMEM_SHARED((NP, D), _f32),
                   pltpu.SemaphoreType.DMA,
                   pltpu.SemaphoreType.DMA,
                   pltpu.SemaphoreType.DMA,
                   pltpu.SemaphoreType.DMA],
)
def _edge_scatter(u_hbm, src_hbm, dst_hbm, zeros2_hbm, out_hbm,
                  si_v, di_v, rows0_v, rows1_v, acc_sh, g0, g1, s0, s1):
    c = lax.axis_index("c")
    s = lax.axis_index("s")
    wid = s * NC + c
    rs = s * RPS
    pltpu.sync_copy(src_hbm.at[wid], si_v)
    pltpu.sync_copy(dst_hbm.at[wid], di_v)
    pltpu.sync_copy(zeros2_hbm.at[pl.ds(rs, RPS)], acc_sh.at[pl.ds(rs, RPS)])
    plsc.subcore_barrier()

    def sidx(c):
        return si_v.at[pl.ds(c * CHE, CHE)]

    pltpu.async_copy(u_hbm.at[sidx(0)], rows0_v, g0)

    def pair(i, carry):
        c0 = 2 * i
        pltpu.async_copy(u_hbm.at[sidx(c0 + 1)], rows1_v, g1)
        pltpu.make_async_copy(u_hbm.at[sidx(c0)], rows0_v, g0).wait()
        pltpu.sync_copy(rows0_v, acc_sh.at[di_v.at[c0]], add=True)

        @pl.when(i < NCHUNK // 2 - 1)
        def _():
            pltpu.async_copy(u_hbm.at[sidx(c0 + 2)], rows0_v, g0)

        pltpu.make_async_copy(u_hbm.at[sidx(c0 + 1)], rows1_v, g1).wait()
        pltpu.sync_copy(rows1_v, acc_sh.at[di_v.at[c0 + 1]], add=True)
        return carry

    lax.fori_loop(0, NCHUNK // 2, pair, 0)
    plsc.subcore_barrier()
    pltpu.sync_copy(acc_sh.at[pl.ds(rs, RPS)], out_hbm.at[c, pl.ds(rs, RPS)])


R = 1024
NBLK = NP // R


def _prep1_body(x_ref, cnt_ref, w_ref, u_ref, dinv_ref):
    i = pl.program_id(0)
    tot = jnp.sum(cnt_ref[...], axis=1, keepdims=True)
    rows = i * R + lax.broadcasted_iota(jnp.int32, (R, 1), 0)
    dinv = jnp.where(rows < N, lax.rsqrt(tot + 1.0), 0.0)
    u_ref[...] = jnp.dot(x_ref[...], w_ref[...],
                         preferred_element_type=_f32) * dinv
    dinv_ref[...] = dinv


def _prep1(x_p, ecntT, W1):
    return pl.pallas_call(
        _prep1_body,
        grid=(NBLK,),
        in_specs=[pl.BlockSpec((R, D), lambda i: (i, 0)),
                  pl.BlockSpec((R, NW), lambda i: (i, 0)),
                  pl.BlockSpec((D, D), lambda i: (0, 0))],
        out_specs=[pl.BlockSpec((R, D), lambda i: (i, 0)),
                   pl.BlockSpec((R, 1), lambda i: (i, 0))],
        out_shape=[jax.ShapeDtypeStruct((NP, D), _f32),
                   jax.ShapeDtypeStruct((NP, 1), _f32)],
    )(x_p, ecntT, W1)


def _mid_body(y_ref, u_ref, dinv_ref, b_ref, w_ref, o_ref):
    ys = jnp.sum(y_ref[...], axis=0)
    dinv = dinv_ref[...]
    h = jnp.maximum((ys + u_ref[...]) * dinv + b_ref[...], 0.0)
    o_ref[...] = jnp.dot(h, w_ref[...], preferred_element_type=_f32) * dinv


def _mid(y, u, dinv, b, W):
    return pl.pallas_call(
        _mid_body,
        grid=(NBLK,),
        in_specs=[pl.BlockSpec((NC, R, D), lambda i: (0, i, 0)),
                  pl.BlockSpec((R, D), lambda i: (i, 0)),
                  pl.BlockSpec((R, 1), lambda i: (i, 0)),
                  pl.BlockSpec((1, D), lambda i: (0, 0)),
                  pl.BlockSpec((D, D), lambda i: (0, 0))],
        out_specs=pl.BlockSpec((R, D), lambda i: (i, 0)),
        out_shape=jax.ShapeDtypeStruct((NP, D), _f32),
    )(y, u, dinv, b, W)


def _tail_body(y_ref, u_ref, dinv_ref, b_ref, batch_ref, wc_ref, bc_ref,
               o_ref, sums_ref, cnt_ref):
    i = pl.program_id(0)
    ys = jnp.sum(y_ref[...], axis=0)
    h = jnp.maximum((ys + u_ref[...]) * dinv_ref[...] + b_ref[...], 0.0)
    cols = lax.broadcasted_iota(jnp.int32, (R, G), 1)
    oh = (batch_ref[...] == cols).astype(_f32)
    dn = (((0,), (0,)), ((), ()))

    @pl.when(i == 0)
    def _():
        sums_ref[...] = jnp.zeros_like(sums_ref)
        cnt_ref[...] = jnp.zeros_like(cnt_ref)

    sums_ref[...] += lax.dot_general(oh, h, dn, preferred_element_type=_f32)
    cnt_ref[...] += lax.dot_general(oh, jnp.ones((R, 1), _f32), dn,
                                    preferred_element_type=_f32)

    @pl.when(i == NBLK - 1)
    def _():
        pooled = sums_ref[...] / jnp.maximum(cnt_ref[...], 1.0)
        o_ref[...] = jnp.dot(pooled, wc_ref[...],
                             preferred_element_type=_f32) + bc_ref[...]


def _tail(y, u, dinv, b, batch_col, Wc, bc):
    return pl.pallas_call(
        _tail_body,
        grid=(NBLK,),
        in_specs=[pl.BlockSpec((NC, R, D), lambda i: (0, i, 0)),
                  pl.BlockSpec((R, D), lambda i: (i, 0)),
                  pl.BlockSpec((R, 1), lambda i: (i, 0)),
                  pl.BlockSpec((1, D), lambda i: (0, 0)),
                  pl.BlockSpec((R, 1), lambda i: (i, 0)),
                  pl.BlockSpec((D, OUT), lambda i: (0, 0)),
                  pl.BlockSpec((1, OUT), lambda i: (0, 0))],
        out_specs=pl.BlockSpec((G, OUT), lambda i: (0, 0)),
        out_shape=jax.ShapeDtypeStruct((G, OUT), _f32),
        scratch_shapes=[pltpu.VMEM((G, D), _f32),
                        pltpu.VMEM((G, 1), _f32)],
    )(y, u, dinv, b, batch_col, Wc, bc)


def kernel(x, edge_index, batch, W1, b1, W2, b2, W3, b3, Wc, bc):
    src = edge_index[0]
    dst = edge_index[1]
    pad_idx = N + (jnp.arange(EP - E, dtype=jnp.int32) % (NP - N))
    src_p = jnp.concatenate([src, pad_idx])
    dst_p = jnp.concatenate([dst, pad_idx])
    src2 = src_p.reshape(NW, EW)
    dst2 = dst_p.reshape(NW, EW)
    dst3 = dst_p.reshape(NW, NCHUNK, CHE)
    batch_p = jnp.concatenate(
        [batch, G + (jnp.arange(NP - N, dtype=jnp.int32) % G)])
    x_p = jnp.zeros((NP, D), _f32).at[:N].set(x)
    zeros1 = jnp.zeros((NP,), _f32)
    zeros2 = jnp.zeros((NP, D), _f32)

    ecnt = _deg(dst2, zeros1)
    u1, dinv = _prep1(x_p, ecnt.T, W1)
    y1 = _edge_scatter(u1, src2, dst3, zeros2)
    u2 = _mid(y1, u1, dinv, b1.reshape(1, D), W2)
    y2 = _edge_scatter(u2, src2, dst3, zeros2)
    u3 = _mid(y2, u2, dinv, b2.reshape(1, D), W3)
    y3 = _edge_scatter(u3, src2, dst3, zeros2)
    return _tail(y3, u3, dinv, b3.reshape(1, D), batch_p.reshape(NP, 1),
                 Wc, bc.reshape(1, OUT))


# speedup vs baseline: 1.3797x; 1.0965x over previous
import functools

import jax
import jax.numpy as jnp
from jax import lax
from jax.experimental import pallas as pl
from jax.experimental.pallas import tpu as pltpu
from jax.experimental.pallas import tpu_sc as plsc

N = 10000
E = 320000
D = 128
OUT = 3
G = 64

NC, NS, L = 2, 16, 16
NW = NC * NS

NP = 10240
EW = 10240
EP = NW * EW
CH = 128
NCHUNK_DEG = EW // CH
CHE = 128
NCHUNK = EW // CHE
RPS = NP // NS
BG = 128
BPW = NP // NW
CHP = 80

_mesh = plsc.VectorSubcoreMesh(core_axis_name="c", subcore_axis_name="s")
_f32 = jnp.float32


@functools.partial(
    pl.kernel,
    out_type=jax.ShapeDtypeStruct((NW, NP), _f32),
    mesh=_mesh,
    scratch_types=[pltpu.VMEM((NP,), _f32),
                   pltpu.VMEM((EW,), jnp.int32)],
    compiler_params=pltpu.CompilerParams(needs_layout_passes=False),
)
def _deg(dst_hbm, zeros1_hbm, ecnt_hbm, cnt_v, idx_v):
    c = lax.axis_index("c")
    s = lax.axis_index("s")
    wid = s * NC + c
    pltpu.sync_copy(zeros1_hbm, cnt_v)
    pltpu.sync_copy(dst_hbm.at[wid], idx_v)
    ones = jnp.full((L,), 1.0, _f32)

    def chunk(i, carry):
        for k in range(CH // L):
            plsc.addupdate_scatter(
                cnt_v, [idx_v[pl.ds(i * CH + k * L, L)]], ones)
        return carry

    lax.fori_loop(0, NCHUNK_DEG, chunk, 0)
    pltpu.sync_copy(cnt_v, ecnt_hbm.at[wid])


@functools.partial(
    pl.kernel,
    out_type=jax.ShapeDtypeStruct((NC, NP, D), _f32),
    mesh=_mesh,
    scratch_types=[pltpu.VMEM((EW,), jnp.int32),
                   pltpu.VMEM((1, CHE), jnp.int32),
                   pltpu.VMEM((1, CHE), jnp.int32),
                   pltpu.VMEM((CHE, D), _f32),
                   pltpu.VMEM((CHE, D), _f32),
                   pltpu.VMEM_SHARED((NP, D), _f32),
                   pltpu.SemaphoreType.DMA,
                   pltpu.SemaphoreType.DMA,
                   pltpu.SemaphoreType.DMA,
                   pltpu.SemaphoreType.DMA],
)
def _edge_scatter(u_hbm, src_hbm, dst_hbm, zeros2_hbm, out_hbm,
                  si_v, di0_v, di1_v, rows0_v, rows1_v, acc_sh,
                  g0, g1, d0, d1):
    c = lax.axis_index("c")
    s = lax.axis_index("s")
    wid = s * NC + c
    rs = s * RPS
    pltpu.sync_copy(src_hbm.at[wid], si_v)
    pltpu.sync_copy(zeros2_hbm.at[pl.ds(rs, RPS)], acc_sh.at[pl.ds(rs, RPS)])
    plsc.subcore_barrier()

    def sidx(c):
        return si_v.at[pl.ds(c * CHE, CHE)]

    pltpu.async_copy(dst_hbm.at[wid, 0], di0_v, d0)
    pltpu.async_copy(dst_hbm.at[wid, 1], di1_v, d1)
    pltpu.async_copy(u_hbm.at[sidx(0)], rows0_v, g0)

    def pair(i, carry):
        c0 = 2 * i
        pltpu.async_copy(u_hbm.at[sidx(c0 + 1)], rows1_v, g1)
        pltpu.make_async_copy(u_hbm.at[sidx(c0)], rows0_v, g0).wait()
        pltpu.make_async_copy(dst_hbm.at[wid, 0], di0_v, d0).wait()
        pltpu.sync_copy(rows0_v, acc_sh.at[di0_v.at[0]], add=True)

        @pl.when(i < NCHUNK // 2 - 1)
        def _():
            pltpu.async_copy(u_hbm.at[sidx(c0 + 2)], rows0_v, g0)
            pltpu.async_copy(dst_hbm.at[wid, c0 + 2], di0_v, d0)

        pltpu.make_async_copy(u_hbm.at[sidx(c0 + 1)], rows1_v, g1).wait()
        pltpu.make_async_copy(dst_hbm.at[wid, 1], di1_v, d1).wait()
        pltpu.sync_copy(rows1_v, acc_sh.at[di1_v.at[0]], add=True)

        @pl.when(i < NCHUNK // 2 - 1)
        def _():
            pltpu.async_copy(dst_hbm.at[wid, c0 + 3], di1_v, d1)

        return carry

    lax.fori_loop(0, NCHUNK // 2, pair, 0)
    plsc.subcore_barrier()
    pltpu.sync_copy(acc_sh.at[pl.ds(rs, RPS)], out_hbm.at[c, pl.ds(rs, RPS)])


R = 1024
NBLK = NP // R


def _prep1_body(x_ref, cnt_ref, w_ref, u_ref, dinv_ref):
    i = pl.program_id(0)
    tot = jnp.sum(cnt_ref[...], axis=1, keepdims=True)
    rows = i * R + lax.broadcasted_iota(jnp.int32, (R, 1), 0)
    dinv = jnp.where(rows < N, lax.rsqrt(tot + 1.0), 0.0)
    u_ref[...] = jnp.dot(x_ref[...], w_ref[...],
                         preferred_element_type=_f32) * dinv
    dinv_ref[...] = dinv


def _prep1(x_p, ecntT, W1):
    return pl.pallas_call(
        _prep1_body,
        grid=(NBLK,),
        in_specs=[pl.BlockSpec((R, D), lambda i: (i, 0)),
                  pl.BlockSpec((R, NW), lambda i: (i, 0)),
                  pl.BlockSpec((D, D), lambda i: (0, 0))],
        out_specs=[pl.BlockSpec((R, D), lambda i: (i, 0)),
                   pl.BlockSpec((R, 1), lambda i: (i, 0))],
        out_shape=[jax.ShapeDtypeStruct((NP, D), _f32),
                   jax.ShapeDtypeStruct((NP, 1), _f32)],
    )(x_p, ecntT, W1)


def _mid_body(y_ref, u_ref, dinv_ref, b_ref, w_ref, o_ref):
    ys = jnp.sum(y_ref[...], axis=0)
    dinv = dinv_ref[...]
    h = jnp.maximum((ys + u_ref[...]) * dinv + b_ref[...], 0.0)
    o_ref[...] = jnp.dot(h, w_ref[...], preferred_element_type=_f32) * dinv


def _mid(y, u, dinv, b, W):
    return pl.pallas_call(
        _mid_body,
        grid=(NBLK,),
        in_specs=[pl.BlockSpec((NC, R, D), lambda i: (0, i, 0)),
                  pl.BlockSpec((R, D), lambda i: (i, 0)),
                  pl.BlockSpec((R, 1), lambda i: (i, 0)),
                  pl.BlockSpec((1, D), lambda i: (0, 0)),
                  pl.BlockSpec((D, D), lambda i: (0, 0))],
        out_specs=pl.BlockSpec((R, D), lambda i: (i, 0)),
        out_shape=jax.ShapeDtypeStruct((NP, D), _f32),
    )(y, u, dinv, b, W)


def _tail_body(y_ref, u_ref, dinv_ref, b_ref, batch_ref, wc_ref, bc_ref,
               o_ref, sums_ref, cnt_ref):
    i = pl.program_id(0)
    ys = jnp.sum(y_ref[...], axis=0)
    h = jnp.maximum((ys + u_ref[...]) * dinv_ref[...] + b_ref[...], 0.0)
    cols = lax.broadcasted_iota(jnp.int32, (R, G), 1)
    oh = (batch_ref[...] == cols).astype(_f32)
    dn = (((0,), (0,)), ((), ()))

    @pl.when(i == 0)
    def _():
        sums_ref[...] = jnp.zeros_like(sums_ref)
        cnt_ref[...] = jnp.zeros_like(cnt_ref)

    sums_ref[...] += lax.dot_general(oh, h, dn, preferred_element_type=_f32)
    cnt_ref[...] += lax.dot_general(oh, jnp.ones((R, 1), _f32), dn,
                                    preferred_element_type=_f32)

    @pl.when(i == NBLK - 1)
    def _():
        pooled = sums_ref[...] / jnp.maximum(cnt_ref[...], 1.0)
        o_ref[...] = jnp.dot(pooled, wc_ref[...],
                             preferred_element_type=_f32) + bc_ref[...]


def _tail(y, u, dinv, b, batch_col, Wc, bc):
    return pl.pallas_call(
        _tail_body,
        grid=(NBLK,),
        in_specs=[pl.BlockSpec((NC, R, D), lambda i: (0, i, 0)),
                  pl.BlockSpec((R, D), lambda i: (i, 0)),
                  pl.BlockSpec((R, 1), lambda i: (i, 0)),
                  pl.BlockSpec((1, D), lambda i: (0, 0)),
                  pl.BlockSpec((R, 1), lambda i: (i, 0)),
                  pl.BlockSpec((D, OUT), lambda i: (0, 0)),
                  pl.BlockSpec((1, OUT), lambda i: (0, 0))],
        out_specs=pl.BlockSpec((G, OUT), lambda i: (0, 0)),
        out_shape=jax.ShapeDtypeStruct((G, OUT), _f32),
        scratch_shapes=[pltpu.VMEM((G, D), _f32),
                        pltpu.VMEM((G, 1), _f32)],
    )(y, u, dinv, b, batch_col, Wc, bc)


def kernel(x, edge_index, batch, W1, b1, W2, b2, W3, b3, Wc, bc):
    src = edge_index[0]
    dst = edge_index[1]
    pad_idx = N + (jnp.arange(EP - E, dtype=jnp.int32) % (NP - N))
    src_p = jnp.concatenate([src, pad_idx])
    dst_p = jnp.concatenate([dst, pad_idx])
    src2 = src_p.reshape(NW, EW)
    dst2 = dst_p.reshape(NW, EW)
    dst4 = dst_p.reshape(NW, NCHUNK, 1, CHE)
    batch_p = jnp.concatenate(
        [batch, G + (jnp.arange(NP - N, dtype=jnp.int32) % G)])
    x_p = jnp.zeros((NP, D), _f32).at[:N].set(x)
    zeros1 = jnp.zeros((NP,), _f32)
    zeros2 = jnp.zeros((NP, D), _f32)

    ecnt = _deg(dst2, zeros1)
    u1, dinv = _prep1(x_p, ecnt.T, W1)
    y1 = _edge_scatter(u1, src2, dst4, zeros2)
    u2 = _mid(y1, u1, dinv, b1.reshape(1, D), W2)
    y2 = _edge_scatter(u2, src2, dst4, zeros2)
    u3 = _mid(y2, u2, dinv, b2.reshape(1, D), W3)
    y3 = _edge_scatter(u3, src2, dst4, zeros2)
    return _tail(y3, u3, dinv, b3.reshape(1, D), batch_p.reshape(NP, 1),
                 Wc, bc.reshape(1, OUT))


# speedup vs baseline: 1.4075x; 1.0202x over previous
import functools

import jax
import jax.numpy as jnp
from jax import lax
from jax.experimental import pallas as pl
from jax.experimental.pallas import tpu as pltpu
from jax.experimental.pallas import tpu_sc as plsc

N = 10000
E = 320000
D = 128
OUT = 3
G = 64

NC, NS, L = 2, 16, 16
NW = NC * NS

NP = 10240
EW = 10240
EP = NW * EW
CH = 128
NCHUNK_DEG = EW // CH
CHE = 128
NCHUNK = EW // CHE
RPS = NP // NS
BG = 128
BPW = NP // NW
CHP = 80

_mesh = plsc.VectorSubcoreMesh(core_axis_name="c", subcore_axis_name="s")
_f32 = jnp.float32


@functools.partial(
    pl.kernel,
    out_type=jax.ShapeDtypeStruct((NW, NP), _f32),
    mesh=_mesh,
    scratch_types=[pltpu.VMEM((NP,), _f32),
                   pltpu.VMEM((EW,), jnp.int32)],
    compiler_params=pltpu.CompilerParams(needs_layout_passes=False),
)
def _deg(dst_hbm, zeros1_hbm, ecnt_hbm, cnt_v, idx_v):
    c = lax.axis_index("c")
    s = lax.axis_index("s")
    wid = s * NC + c
    pltpu.sync_copy(zeros1_hbm, cnt_v)
    pltpu.sync_copy(dst_hbm.at[wid], idx_v)
    ones = jnp.full((L,), 1.0, _f32)

    def chunk(i, carry):
        for k in range(CH // L):
            plsc.addupdate_scatter(
                cnt_v, [idx_v[pl.ds(i * CH + k * L, L)]], ones)
        return carry

    lax.fori_loop(0, NCHUNK_DEG, chunk, 0)
    pltpu.sync_copy(cnt_v, ecnt_hbm.at[wid])


@functools.partial(
    pl.kernel,
    out_type=jax.ShapeDtypeStruct((NC, NP, D), _f32),
    mesh=_mesh,
    scratch_types=[pltpu.VMEM((EW,), jnp.int32),
                   pltpu.VMEM((1, CHE), jnp.int32),
                   pltpu.VMEM((1, CHE), jnp.int32),
                   pltpu.VMEM((CHE, D), _f32),
                   pltpu.VMEM((CHE, D), _f32),
                   pltpu.VMEM_SHARED((NP, D), _f32),
                   pltpu.SemaphoreType.DMA,
                   pltpu.SemaphoreType.DMA,
                   pltpu.SemaphoreType.DMA,
                   pltpu.SemaphoreType.DMA],
)
def _edge_scatter(u_hbm, src_hbm, dst_hbm, zeros2_hbm, out_hbm,
                  si_v, di0_v, di1_v, rows0_v, rows1_v, acc_sh,
                  g0, g1, d0, d1):
    c = lax.axis_index("c")
    s = lax.axis_index("s")
    wid = s * NC + c
    rs = s * RPS
    pltpu.sync_copy(src_hbm.at[wid], si_v)
    pltpu.sync_copy(zeros2_hbm.at[pl.ds(rs, RPS)], acc_sh.at[pl.ds(rs, RPS)])
    plsc.subcore_barrier()

    def sidx(c):
        return si_v.at[pl.ds(c * CHE, CHE)]

    pltpu.async_copy(dst_hbm.at[wid, 0], di0_v, d0)
    pltpu.async_copy(dst_hbm.at[wid, 1], di1_v, d1)
    pltpu.async_copy(u_hbm.at[sidx(0)], rows0_v, g0)

    def pair(i, carry):
        c0 = 2 * i
        pltpu.async_copy(u_hbm.at[sidx(c0 + 1)], rows1_v, g1)
        pltpu.make_async_copy(u_hbm.at[sidx(c0)], rows0_v, g0).wait()
        pltpu.make_async_copy(dst_hbm.at[wid, 0], di0_v, d0).wait()
        pltpu.sync_copy(rows0_v, acc_sh.at[di0_v.at[0]], add=True)

        @pl.when(i < NCHUNK // 2 - 1)
        def _():
            pltpu.async_copy(u_hbm.at[sidx(c0 + 2)], rows0_v, g0)
            pltpu.async_copy(dst_hbm.at[wid, c0 + 2], di0_v, d0)

        pltpu.make_async_copy(u_hbm.at[sidx(c0 + 1)], rows1_v, g1).wait()
        pltpu.make_async_copy(dst_hbm.at[wid, 1], di1_v, d1).wait()
        pltpu.sync_copy(rows1_v, acc_sh.at[di1_v.at[0]], add=True)

        @pl.when(i < NCHUNK // 2 - 1)
        def _():
            pltpu.async_copy(dst_hbm.at[wid, c0 + 3], di1_v, d1)

        return carry

    lax.fori_loop(0, NCHUNK // 2, pair, 0)
    plsc.subcore_barrier()
    pltpu.sync_copy(acc_sh.at[pl.ds(rs, RPS)], out_hbm.at[c, pl.ds(rs, RPS)])


R = 2048
NBLK = NP // R


def _prep1_body(x_ref, cnt_ref, w_ref, u_ref, dinv_ref):
    i = pl.program_id(0)
    tot = jnp.sum(cnt_ref[...], axis=1, keepdims=True)
    rows = i * R + lax.broadcasted_iota(jnp.int32, (R, 1), 0)
    dinv = jnp.where(rows < N, lax.rsqrt(tot + 1.0), 0.0)
    u_ref[...] = jnp.dot(x_ref[...], w_ref[...],
                         preferred_element_type=_f32) * dinv
    dinv_ref[...] = dinv


def _prep1(x_p, ecntT, W1):
    return pl.pallas_call(
        _prep1_body,
        grid=(NBLK,),
        in_specs=[pl.BlockSpec((R, D), lambda i: (i, 0)),
                  pl.BlockSpec((R, NW), lambda i: (i, 0)),
                  pl.BlockSpec((D, D), lambda i: (0, 0))],
        out_specs=[pl.BlockSpec((R, D), lambda i: (i, 0)),
                   pl.BlockSpec((R, 1), lambda i: (i, 0))],
        out_shape=[jax.ShapeDtypeStruct((NP, D), _f32),
                   jax.ShapeDtypeStruct((NP, 1), _f32)],
    )(x_p, ecntT, W1)


def _mid_body(y_ref, u_ref, dinv_ref, b_ref, w_ref, o_ref):
    ys = jnp.sum(y_ref[...], axis=0)
    dinv = dinv_ref[...]
    h = jnp.maximum((ys + u_ref[...]) * dinv + b_ref[...], 0.0)
    o_ref[...] = jnp.dot(h, w_ref[...], preferred_element_type=_f32) * dinv


def _mid(y, u, dinv, b, W):
    return pl.pallas_call(
        _mid_body,
        grid=(NBLK,),
        in_specs=[pl.BlockSpec((NC, R, D), lambda i: (0, i, 0)),
                  pl.BlockSpec((R, D), lambda i: (i, 0)),
                  pl.BlockSpec((R, 1), lambda i: (i, 0)),
                  pl.BlockSpec((1, D), lambda i: (0, 0)),
                  pl.BlockSpec((D, D), lambda i: (0, 0))],
        out_specs=pl.BlockSpec((R, D), lambda i: (i, 0)),
        out_shape=jax.ShapeDtypeStruct((NP, D), _f32),
    )(y, u, dinv, b, W)


def _tail_body(y_ref, u_ref, dinv_ref, b_ref, batch_ref, wc_ref, bc_ref,
               o_ref, sums_ref, cnt_ref):
    i = pl.program_id(0)
    ys = jnp.sum(y_ref[...], axis=0)
    h = jnp.maximum((ys + u_ref[...]) * dinv_ref[...] + b_ref[...], 0.0)
    cols = lax.broadcasted_iota(jnp.int32, (R, G), 1)
    oh = (batch_ref[...] == cols).astype(_f32)
    dn = (((0,), (0,)), ((), ()))

    @pl.when(i == 0)
    def _():
        sums_ref[...] = jnp.zeros_like(sums_ref)
        cnt_ref[...] = jnp.zeros_like(cnt_ref)

    sums_ref[...] += lax.dot_general(oh, h, dn, preferred_element_type=_f32)
    cnt_ref[...] += lax.dot_general(oh, jnp.ones((R, 1), _f32), dn,
                                    preferred_element_type=_f32)

    @pl.when(i == NBLK - 1)
    def _():
        pooled = sums_ref[...] / jnp.maximum(cnt_ref[...], 1.0)
        o_ref[...] = jnp.dot(pooled, wc_ref[...],
                             preferred_element_type=_f32) + bc_ref[...]


def _tail(y, u, dinv, b, batch_col, Wc, bc):
    return pl.pallas_call(
        _tail_body,
        grid=(NBLK,),
        in_specs=[pl.BlockSpec((NC, R, D), lambda i: (0, i, 0)),
                  pl.BlockSpec((R, D), lambda i: (i, 0)),
                  pl.BlockSpec((R, 1), lambda i: (i, 0)),
                  pl.BlockSpec((1, D), lambda i: (0, 0)),
                  pl.BlockSpec((R, 1), lambda i: (i, 0)),
                  pl.BlockSpec((D, OUT), lambda i: (0, 0)),
                  pl.BlockSpec((1, OUT), lambda i: (0, 0))],
        out_specs=pl.BlockSpec((G, OUT), lambda i: (0, 0)),
        out_shape=jax.ShapeDtypeStruct((G, OUT), _f32),
        scratch_shapes=[pltpu.VMEM((G, D), _f32),
                        pltpu.VMEM((G, 1), _f32)],
    )(y, u, dinv, b, batch_col, Wc, bc)


def kernel(x, edge_index, batch, W1, b1, W2, b2, W3, b3, Wc, bc):
    src = edge_index[0]
    dst = edge_index[1]
    pad_idx = N + (jnp.arange(EP - E, dtype=jnp.int32) % (NP - N))
    src_p = jnp.concatenate([src, pad_idx])
    dst_p = jnp.concatenate([dst, pad_idx])
    src2 = src_p.reshape(NW, EW)
    dst2 = dst_p.reshape(NW, EW)
    dst4 = dst_p.reshape(NW, NCHUNK, 1, CHE)
    batch_p = jnp.concatenate(
        [batch, G + (jnp.arange(NP - N, dtype=jnp.int32) % G)])
    x_p = jnp.zeros((NP, D), _f32).at[:N].set(x)
    zeros1 = jnp.zeros((NP,), _f32)
    zeros2 = jnp.zeros((NP, D), _f32)

    ecnt = _deg(dst2, zeros1)
    u1, dinv = _prep1(x_p, ecnt.T, W1)
    y1 = _edge_scatter(u1, src2, dst4, zeros2)
    u2 = _mid(y1, u1, dinv, b1.reshape(1, D), W2)
    y2 = _edge_scatter(u2, src2, dst4, zeros2)
    u3 = _mid(y2, u2, dinv, b2.reshape(1, D), W3)
    y3 = _edge_scatter(u3, src2, dst4, zeros2)
    return _tail(y3, u3, dinv, b3.reshape(1, D), batch_p.reshape(NP, 1),
                 Wc, bc.reshape(1, OUT))


# speedup vs baseline: 1.4137x; 1.0044x over previous
import functools

import jax
import jax.numpy as jnp
from jax import lax
from jax.experimental import pallas as pl
from jax.experimental.pallas import tpu as pltpu
from jax.experimental.pallas import tpu_sc as plsc

N = 10000
E = 320000
D = 128
OUT = 3
G = 64

NC, NS, L = 2, 16, 16
NW = NC * NS

NP = 10240
EW = 10240
EP = NW * EW
CH = 128
NCHUNK_DEG = EW // CH
CHE = 128
NCHUNK = EW // CHE
RPS = NP // NS
BG = 128
BPW = NP // NW
CHP = 80

_mesh = plsc.VectorSubcoreMesh(core_axis_name="c", subcore_axis_name="s")
_f32 = jnp.float32


@functools.partial(
    pl.kernel,
    out_type=jax.ShapeDtypeStruct((NW, NP), _f32),
    mesh=_mesh,
    scratch_types=[pltpu.VMEM((NP,), _f32),
                   pltpu.VMEM((EW,), jnp.int32)],
    compiler_params=pltpu.CompilerParams(needs_layout_passes=False),
)
def _deg(dst_hbm, zeros1_hbm, ecnt_hbm, cnt_v, idx_v):
    c = lax.axis_index("c")
    s = lax.axis_index("s")
    wid = s * NC + c
    pltpu.sync_copy(zeros1_hbm, cnt_v)
    pltpu.sync_copy(dst_hbm.at[wid], idx_v)
    ones = jnp.full((L,), 1.0, _f32)

    def chunk(i, carry):
        for k in range(CH // L):
            plsc.addupdate_scatter(
                cnt_v, [idx_v[pl.ds(i * CH + k * L, L)]], ones)
        return carry

    lax.fori_loop(0, NCHUNK_DEG, chunk, 0)
    pltpu.sync_copy(cnt_v, ecnt_hbm.at[wid])


@functools.partial(
    pl.kernel,
    out_type=jax.ShapeDtypeStruct((NC, NP, D), _f32),
    mesh=_mesh,
    scratch_types=[pltpu.VMEM((EW,), jnp.int32),
                   pltpu.VMEM((1, CHE), jnp.int32),
                   pltpu.VMEM((1, CHE), jnp.int32),
                   pltpu.VMEM((CHE, D), _f32),
                   pltpu.VMEM((CHE, D), _f32),
                   pltpu.VMEM_SHARED((NP, D), _f32),
                   pltpu.SemaphoreType.DMA,
                   pltpu.SemaphoreType.DMA,
                   pltpu.SemaphoreType.DMA,
                   pltpu.SemaphoreType.DMA],
)
def _edge_scatter(u_hbm, src_hbm, dst_hbm, zeros2_hbm, out_hbm,
                  si_v, di0_v, di1_v, rows0_v, rows1_v, acc_sh,
                  g0, g1, d0, d1):
    c = lax.axis_index("c")
    s = lax.axis_index("s")
    wid = s * NC + c
    rs = s * RPS
    pltpu.sync_copy(src_hbm.at[wid], si_v)
    pltpu.sync_copy(zeros2_hbm.at[pl.ds(rs, RPS)], acc_sh.at[pl.ds(rs, RPS)])
    plsc.subcore_barrier()

    def sidx(c):
        return si_v.at[pl.ds(c * CHE, CHE)]

    pltpu.async_copy(dst_hbm.at[wid, 0], di0_v, d0)
    pltpu.async_copy(dst_hbm.at[wid, 1], di1_v, d1)
    pltpu.async_copy(u_hbm.at[sidx(0)], rows0_v, g0)

    def pair(i, carry):
        c0 = 2 * i
        pltpu.async_copy(u_hbm.at[sidx(c0 + 1)], rows1_v, g1)
        pltpu.make_async_copy(u_hbm.at[sidx(c0)], rows0_v, g0).wait()
        pltpu.make_async_copy(dst_hbm.at[wid, 0], di0_v, d0).wait()
        pltpu.sync_copy(rows0_v, acc_sh.at[di0_v.at[0]], add=True)

        @pl.when(i < NCHUNK // 2 - 1)
        def _():
            pltpu.async_copy(u_hbm.at[sidx(c0 + 2)], rows0_v, g0)
            pltpu.async_copy(dst_hbm.at[wid, c0 + 2], di0_v, d0)

        pltpu.make_async_copy(u_hbm.at[sidx(c0 + 1)], rows1_v, g1).wait()
        pltpu.make_async_copy(dst_hbm.at[wid, 1], di1_v, d1).wait()
        pltpu.sync_copy(rows1_v, acc_sh.at[di1_v.at[0]], add=True)

        @pl.when(i < NCHUNK // 2 - 1)
        def _():
            pltpu.async_copy(dst_hbm.at[wid, c0 + 3], di1_v, d1)

        return carry

    lax.fori_loop(0, NCHUNK // 2, pair, 0)
    plsc.subcore_barrier()
    pltpu.sync_copy(acc_sh.at[pl.ds(rs, RPS)], out_hbm.at[c, pl.ds(rs, RPS)])


R = 2560
NBLK = NP // R


def _prep1_body(x_ref, cnt_ref, w_ref, u_ref, dinv_ref):
    i = pl.program_id(0)
    tot = jnp.sum(cnt_ref[...], axis=1, keepdims=True)
    rows = i * R + lax.broadcasted_iota(jnp.int32, (R, 1), 0)
    dinv = jnp.where(rows < N, lax.rsqrt(tot + 1.0), 0.0)
    u_ref[...] = jnp.dot(x_ref[...], w_ref[...],
                         preferred_element_type=_f32) * dinv
    dinv_ref[...] = dinv


def _prep1(x_p, ecntT, W1):
    return pl.pallas_call(
        _prep1_body,
        grid=(NBLK,),
        in_specs=[pl.BlockSpec((R, D), lambda i: (i, 0)),
                  pl.BlockSpec((R, NW), lambda i: (i, 0)),
                  pl.BlockSpec((D, D), lambda i: (0, 0))],
        out_specs=[pl.BlockSpec((R, D), lambda i: (i, 0)),
                   pl.BlockSpec((R, 1), lambda i: (i, 0))],
        out_shape=[jax.ShapeDtypeStruct((NP, D), _f32),
                   jax.ShapeDtypeStruct((NP, 1), _f32)],
    )(x_p, ecntT, W1)


def _mid_body(y_ref, u_ref, dinv_ref, b_ref, w_ref, o_ref):
    ys = jnp.sum(y_ref[...], axis=0)
    dinv = dinv_ref[...]
    h = jnp.maximum((ys + u_ref[...]) * dinv + b_ref[...], 0.0)
    o_ref[...] = jnp.dot(h, w_ref[...], preferred_element_type=_f32) * dinv


def _mid(y, u, dinv, b, W):
    return pl.pallas_call(
        _mid_body,
        grid=(NBLK,),
        in_specs=[pl.BlockSpec((NC, R, D), lambda i: (0, i, 0)),
                  pl.BlockSpec((R, D), lambda i: (i, 0)),
                  pl.BlockSpec((R, 1), lambda i: (i, 0)),
                  pl.BlockSpec((1, D), lambda i: (0, 0)),
                  pl.BlockSpec((D, D), lambda i: (0, 0))],
        out_specs=pl.BlockSpec((R, D), lambda i: (i, 0)),
        out_shape=jax.ShapeDtypeStruct((NP, D), _f32),
    )(y, u, dinv, b, W)


def _tail_body(y_ref, u_ref, dinv_ref, b_ref, batch_ref, wc_ref, bc_ref,
               o_ref, sums_ref, cnt_ref):
    i = pl.program_id(0)
    ys = jnp.sum(y_ref[...], axis=0)
    h = jnp.maximum((ys + u_ref[...]) * dinv_ref[...] + b_ref[...], 0.0)
    cols = lax.broadcasted_iota(jnp.int32, (R, G), 1)
    oh = (batch_ref[...] == cols).astype(_f32)
    dn = (((0,), (0,)), ((), ()))

    @pl.when(i == 0)
    def _():
        sums_ref[...] = jnp.zeros_like(sums_ref)
        cnt_ref[...] = jnp.zeros_like(cnt_ref)

    sums_ref[...] += lax.dot_general(oh, h, dn, preferred_element_type=_f32)
    cnt_ref[...] += lax.dot_general(oh, jnp.ones((R, 1), _f32), dn,
                                    preferred_element_type=_f32)

    @pl.when(i == NBLK - 1)
    def _():
        pooled = sums_ref[...] / jnp.maximum(cnt_ref[...], 1.0)
        o_ref[...] = jnp.dot(pooled, wc_ref[...],
                             preferred_element_type=_f32) + bc_ref[...]


def _tail(y, u, dinv, b, batch_col, Wc, bc):
    return pl.pallas_call(
        _tail_body,
        grid=(NBLK,),
        in_specs=[pl.BlockSpec((NC, R, D), lambda i: (0, i, 0)),
                  pl.BlockSpec((R, D), lambda i: (i, 0)),
                  pl.BlockSpec((R, 1), lambda i: (i, 0)),
                  pl.BlockSpec((1, D), lambda i: (0, 0)),
                  pl.BlockSpec((R, 1), lambda i: (i, 0)),
                  pl.BlockSpec((D, OUT), lambda i: (0, 0)),
                  pl.BlockSpec((1, OUT), lambda i: (0, 0))],
        out_specs=pl.BlockSpec((G, OUT), lambda i: (0, 0)),
        out_shape=jax.ShapeDtypeStruct((G, OUT), _f32),
        scratch_shapes=[pltpu.VMEM((G, D), _f32),
                        pltpu.VMEM((G, 1), _f32)],
    )(y, u, dinv, b, batch_col, Wc, bc)


def kernel(x, edge_index, batch, W1, b1, W2, b2, W3, b3, Wc, bc):
    src = edge_index[0]
    dst = edge_index[1]
    pad_idx = N + (jnp.arange(EP - E, dtype=jnp.int32) % (NP - N))
    src_p = jnp.concatenate([src, pad_idx])
    dst_p = jnp.concatenate([dst, pad_idx])
    src2 = src_p.reshape(NW, EW)
    dst2 = dst_p.reshape(NW, EW)
    dst4 = dst_p.reshape(NW, NCHUNK, 1, CHE)
    batch_p = jnp.concatenate(
        [batch, G + (jnp.arange(NP - N, dtype=jnp.int32) % G)])
    x_p = jnp.zeros((NP, D), _f32).at[:N].set(x)
    zeros1 = jnp.zeros((NP,), _f32)
    zeros2 = jnp.zeros((NP, D), _f32)

    ecnt = _deg(dst2, zeros1)
    u1, dinv = _prep1(x_p, ecnt.T, W1)
    y1 = _edge_scatter(u1, src2, dst4, zeros2)
    u2 = _mid(y1, u1, dinv, b1.reshape(1, D), W2)
    y2 = _edge_scatter(u2, src2, dst4, zeros2)
    u3 = _mid(y2, u2, dinv, b2.reshape(1, D), W3)
    y3 = _edge_scatter(u3, src2, dst4, zeros2)
    return _tail(y3, u3, dinv, b3.reshape(1, D), batch_p.reshape(NP, 1),
                 Wc, bc.reshape(1, OUT))


# speedup vs baseline: 1.4511x; 1.0265x over previous
import functools

import jax
import jax.numpy as jnp
from jax import lax
from jax.experimental import pallas as pl
from jax.experimental.pallas import tpu as pltpu
from jax.experimental.pallas import tpu_sc as plsc

N = 10000
E = 320000
D = 128
OUT = 3
G = 64

NC, NS, L = 2, 16, 16
NW = NC * NS

NP = 10240
EW = 10240
EP = NW * EW
CH = 128
NCHUNK_DEG = EW // CH
CHE = 128
NCHUNK = EW // CHE
RPS = NP // NS
BG = 128
BPW = NP // NW
CHP = 80

_mesh = plsc.VectorSubcoreMesh(core_axis_name="c", subcore_axis_name="s")
_f32 = jnp.float32


@functools.partial(
    pl.kernel,
    out_type=jax.ShapeDtypeStruct((NW, NP), _f32),
    mesh=_mesh,
    scratch_types=[pltpu.VMEM((NP,), _f32),
                   pltpu.VMEM((EW,), jnp.int32)],
    compiler_params=pltpu.CompilerParams(needs_layout_passes=False),
)
def _deg(dst_hbm, zeros1_hbm, ecnt_hbm, cnt_v, idx_v):
    c = lax.axis_index("c")
    s = lax.axis_index("s")
    wid = s * NC + c
    pltpu.sync_copy(zeros1_hbm, cnt_v)
    pltpu.sync_copy(dst_hbm.at[wid], idx_v)
    ones = jnp.full((L,), 1.0, _f32)

    def chunk(i, carry):
        for k in range(CH // L):
            plsc.addupdate_scatter(
                cnt_v, [idx_v[pl.ds(i * CH + k * L, L)]], ones)
        return carry

    lax.fori_loop(0, NCHUNK_DEG, chunk, 0)
    pltpu.sync_copy(cnt_v, ecnt_hbm.at[wid])


@functools.partial(
    pl.kernel,
    out_type=jax.ShapeDtypeStruct((NC, NP, D), _f32),
    mesh=_mesh,
    scratch_types=[pltpu.VMEM((EW,), jnp.int32),
                   pltpu.VMEM((1, CHE), jnp.int32),
                   pltpu.VMEM((1, CHE), jnp.int32),
                   pltpu.VMEM((CHE, D), _f32),
                   pltpu.VMEM((CHE, D), _f32),
                   pltpu.VMEM_SHARED((NP, D), _f32),
                   pltpu.SemaphoreType.DMA,
                   pltpu.SemaphoreType.DMA,
                   pltpu.SemaphoreType.DMA,
                   pltpu.SemaphoreType.DMA],
)
def _edge_scatter(u_hbm, src_hbm, dst_hbm, zeros2_hbm, out_hbm,
                  si_v, di0_v, di1_v, rows0_v, rows1_v, acc_sh,
                  g0, g1, d0, d1):
    c = lax.axis_index("c")
    s = lax.axis_index("s")
    wid = s * NC + c
    rs = s * RPS
    pltpu.sync_copy(src_hbm.at[wid], si_v)
    pltpu.sync_copy(zeros2_hbm.at[pl.ds(rs, RPS)], acc_sh.at[pl.ds(rs, RPS)])
    plsc.subcore_barrier()

    def sidx(c):
        return si_v.at[pl.ds(c * CHE, CHE)]

    pltpu.async_copy(dst_hbm.at[wid, 0], di0_v, d0)
    pltpu.async_copy(dst_hbm.at[wid, 1], di1_v, d1)
    pltpu.async_copy(u_hbm.at[sidx(0)], rows0_v, g0)

    def pair(i, carry):
        c0 = 2 * i
        pltpu.async_copy(u_hbm.at[sidx(c0 + 1)], rows1_v, g1)
        pltpu.make_async_copy(u_hbm.at[sidx(c0)], rows0_v, g0).wait()
        pltpu.make_async_copy(dst_hbm.at[wid, 0], di0_v, d0).wait()
        pltpu.sync_copy(rows0_v, acc_sh.at[di0_v.at[0]], add=True)

        @pl.when(i < NCHUNK // 2 - 1)
        def _():
            pltpu.async_copy(u_hbm.at[sidx(c0 + 2)], rows0_v, g0)
            pltpu.async_copy(dst_hbm.at[wid, c0 + 2], di0_v, d0)

        pltpu.make_async_copy(u_hbm.at[sidx(c0 + 1)], rows1_v, g1).wait()
        pltpu.make_async_copy(dst_hbm.at[wid, 1], di1_v, d1).wait()
        pltpu.sync_copy(rows1_v, acc_sh.at[di1_v.at[0]], add=True)

        @pl.when(i < NCHUNK // 2 - 1)
        def _():
            pltpu.async_copy(dst_hbm.at[wid, c0 + 3], di1_v, d1)

        return carry

    lax.fori_loop(0, NCHUNK // 2, pair, 0)
    plsc.subcore_barrier()
    pltpu.sync_copy(acc_sh.at[pl.ds(rs, RPS)], out_hbm.at[c, pl.ds(rs, RPS)])


R = 2560
NBLK = NP // R


def _prep1_body(x_ref, cnt_ref, w_ref, u_ref, dinv_ref):
    i = pl.program_id(0)
    dn = (((0,), (0,)), ((), ()))
    tot = lax.dot_general(cnt_ref[...], jnp.ones((NW, 1), _f32), dn,
                          preferred_element_type=_f32)
    rows = i * R + lax.broadcasted_iota(jnp.int32, (R, 1), 0)
    dinv = jnp.where(rows < N, lax.rsqrt(tot + 1.0), 0.0)
    u_ref[...] = jnp.where(
        rows < N,
        jnp.dot(x_ref[...], w_ref[...], preferred_element_type=_f32) * dinv,
        0.0)
    dinv_ref[...] = dinv


def _prep1(x, ecnt, W1):
    return pl.pallas_call(
        _prep1_body,
        grid=(NBLK,),
        in_specs=[pl.BlockSpec((R, D), lambda i: (i, 0)),
                  pl.BlockSpec((NW, R), lambda i: (0, i)),
                  pl.BlockSpec((D, D), lambda i: (0, 0))],
        out_specs=[pl.BlockSpec((R, D), lambda i: (i, 0)),
                   pl.BlockSpec((R, 1), lambda i: (i, 0))],
        out_shape=[jax.ShapeDtypeStruct((NP, D), _f32),
                   jax.ShapeDtypeStruct((NP, 1), _f32)],
    )(x, ecnt, W1)


def _mid_body(y_ref, u_ref, dinv_ref, b_ref, w_ref, o_ref):
    ys = jnp.sum(y_ref[...], axis=0)
    dinv = dinv_ref[...]
    h = jnp.maximum((ys + u_ref[...]) * dinv + b_ref[...], 0.0)
    o_ref[...] = jnp.dot(h, w_ref[...], preferred_element_type=_f32) * dinv


def _mid(y, u, dinv, b, W):
    return pl.pallas_call(
        _mid_body,
        grid=(NBLK,),
        in_specs=[pl.BlockSpec((NC, R, D), lambda i: (0, i, 0)),
                  pl.BlockSpec((R, D), lambda i: (i, 0)),
                  pl.BlockSpec((R, 1), lambda i: (i, 0)),
                  pl.BlockSpec((1, D), lambda i: (0, 0)),
                  pl.BlockSpec((D, D), lambda i: (0, 0))],
        out_specs=pl.BlockSpec((R, D), lambda i: (i, 0)),
        out_shape=jax.ShapeDtypeStruct((NP, D), _f32),
    )(y, u, dinv, b, W)


def _tail_body(y_ref, u_ref, dinv_ref, b_ref, batch_ref, wc_ref, bc_ref,
               o_ref, sums_ref, cnt_ref):
    i = pl.program_id(0)
    ys = jnp.sum(y_ref[...], axis=0)
    h = jnp.maximum((ys + u_ref[...]) * dinv_ref[...] + b_ref[...], 0.0)
    cols = lax.broadcasted_iota(jnp.int32, (R, G), 1)
    rows = i * R + lax.broadcasted_iota(jnp.int32, (R, 1), 0)
    oh = ((batch_ref[...] == cols) & (rows < N)).astype(_f32)
    dn = (((0,), (0,)), ((), ()))

    @pl.when(i == 0)
    def _():
        sums_ref[...] = jnp.zeros_like(sums_ref)
        cnt_ref[...] = jnp.zeros_like(cnt_ref)

    sums_ref[...] += lax.dot_general(oh, h, dn, preferred_element_type=_f32)
    cnt_ref[...] += lax.dot_general(oh, jnp.ones((R, 1), _f32), dn,
                                    preferred_element_type=_f32)

    @pl.when(i == NBLK - 1)
    def _():
        pooled = sums_ref[...] / jnp.maximum(cnt_ref[...], 1.0)
        o_ref[...] = jnp.dot(pooled, wc_ref[...],
                             preferred_element_type=_f32) + bc_ref[...]


def _tail(y, u, dinv, b, batch_col, Wc, bc):
    return pl.pallas_call(
        _tail_body,
        grid=(NBLK,),
        in_specs=[pl.BlockSpec((NC, R, D), lambda i: (0, i, 0)),
                  pl.BlockSpec((R, D), lambda i: (i, 0)),
                  pl.BlockSpec((R, 1), lambda i: (i, 0)),
                  pl.BlockSpec((1, D), lambda i: (0, 0)),
                  pl.BlockSpec((R, 1), lambda i: (i, 0)),
                  pl.BlockSpec((D, OUT), lambda i: (0, 0)),
                  pl.BlockSpec((1, OUT), lambda i: (0, 0))],
        out_specs=pl.BlockSpec((G, OUT), lambda i: (0, 0)),
        out_shape=jax.ShapeDtypeStruct((G, OUT), _f32),
        scratch_shapes=[pltpu.VMEM((G, D), _f32),
                        pltpu.VMEM((G, 1), _f32)],
    )(y, u, dinv, b, batch_col, Wc, bc)


def kernel(x, edge_index, batch, W1, b1, W2, b2, W3, b3, Wc, bc):
    src = edge_index[0]
    dst = edge_index[1]
    pad_idx = N + (jnp.arange(EP - E, dtype=jnp.int32) % (NP - N))
    src_p = jnp.concatenate([src, pad_idx])
    dst_p = jnp.concatenate([dst, pad_idx])
    src2 = src_p.reshape(NW, EW)
    dst2 = dst_p.reshape(NW, EW)
    dst4 = dst_p.reshape(NW, NCHUNK, 1, CHE)
    zeros1 = jnp.zeros((NP,), _f32)
    zeros2 = jnp.zeros((NP, D), _f32)

    ecnt = _deg(dst2, zeros1)
    u1, dinv = _prep1(x, ecnt, W1)
    y1 = _edge_scatter(u1, src2, dst4, zeros2)
    u2 = _mid(y1, u1, dinv, b1.reshape(1, D), W2)
    y2 = _edge_scatter(u2, src2, dst4, zeros2)
    u3 = _mid(y2, u2, dinv, b2.reshape(1, D), W3)
    y3 = _edge_scatter(u3, src2, dst4, zeros2)
    return _tail(y3, u3, dinv, b3.reshape(1, D), batch.reshape(N, 1),
                 Wc, bc.reshape(1, OUT))


# speedup vs baseline: 1.6063x; 1.1070x over previous
import functools

import jax
import jax.numpy as jnp
from jax import lax
from jax.experimental import pallas as pl
from jax.experimental.pallas import tpu as pltpu
from jax.experimental.pallas import tpu_sc as plsc

N = 10000
E = 320000
D = 128
OUT = 3
G = 64

NC, NS, L = 2, 16, 16
NW = NC * NS

NP = 10240
EW = 10240
EP = NW * EW
CH = 128
NCHUNK_DEG = EW // CH
CHE = 64
NCHUNK = EW // CHE
NBUF = 4
RPS = NP // NS
BG = 128
BPW = NP // NW
CHP = 80

_mesh = plsc.VectorSubcoreMesh(core_axis_name="c", subcore_axis_name="s")
_f32 = jnp.float32


@functools.partial(
    pl.kernel,
    out_type=jax.ShapeDtypeStruct((NW, NP), _f32),
    mesh=_mesh,
    scratch_types=[pltpu.VMEM((NP,), _f32),
                   pltpu.VMEM((EW,), jnp.int32)],
    compiler_params=pltpu.CompilerParams(needs_layout_passes=False),
)
def _deg(dst_hbm, zeros1_hbm, ecnt_hbm, cnt_v, idx_v):
    c = lax.axis_index("c")
    s = lax.axis_index("s")
    wid = s * NC + c
    pltpu.sync_copy(zeros1_hbm, cnt_v)
    pltpu.sync_copy(dst_hbm.at[wid], idx_v)
    ones = jnp.full((L,), 1.0, _f32)

    def chunk(i, carry):
        for k in range(CH // L):
            plsc.addupdate_scatter(
                cnt_v, [idx_v[pl.ds(i * CH + k * L, L)]], ones)
        return carry

    lax.fori_loop(0, NCHUNK_DEG, chunk, 0)
    pltpu.sync_copy(cnt_v, ecnt_hbm.at[wid])


@functools.partial(
    pl.kernel,
    out_type=jax.ShapeDtypeStruct((NC, NP, D), _f32),
    mesh=_mesh,
    scratch_types=[pltpu.VMEM((EW,), jnp.int32)]
                  + [pltpu.VMEM((1, CHE), jnp.int32)] * NBUF
                  + [pltpu.VMEM((CHE, D), _f32)] * NBUF
                  + [pltpu.VMEM_SHARED((NP, D), _f32)]
                  + [pltpu.SemaphoreType.DMA] * (2 * NBUF),
)
def _edge_scatter(u_hbm, src_hbm, dst_hbm, zeros2_hbm, out_hbm,
                  si_v, *rest):
    di_v = rest[:NBUF]
    rows_v = rest[NBUF:2 * NBUF]
    acc_sh = rest[2 * NBUF]
    gsem = rest[2 * NBUF + 1:2 * NBUF + 1 + NBUF]
    dsem = rest[2 * NBUF + 1 + NBUF:]
    c = lax.axis_index("c")
    s = lax.axis_index("s")
    wid = s * NC + c
    rs = s * RPS
    pltpu.sync_copy(src_hbm.at[wid], si_v)
    pltpu.sync_copy(zeros2_hbm.at[pl.ds(rs, RPS)], acc_sh.at[pl.ds(rs, RPS)])
    plsc.subcore_barrier()

    def sidx(c):
        return si_v.at[pl.ds(c * CHE, CHE)]

    for k in range(NBUF):
        pltpu.async_copy(dst_hbm.at[wid, k], di_v[k], dsem[k])
        pltpu.async_copy(u_hbm.at[sidx(k)], rows_v[k], gsem[k])

    def quad(i, carry):
        c0 = NBUF * i
        for k in range(NBUF):
            pltpu.make_async_copy(u_hbm.at[sidx(c0 + k)], rows_v[k],
                                  gsem[k]).wait()
            pltpu.make_async_copy(dst_hbm.at[wid, 0], di_v[k],
                                  dsem[k]).wait()
            pltpu.sync_copy(rows_v[k], acc_sh.at[di_v[k].at[0]], add=True)

            @pl.when(i < NCHUNK // NBUF - 1)
            def _():
                pltpu.async_copy(dst_hbm.at[wid, c0 + k + NBUF], di_v[k],
                                 dsem[k])
                pltpu.async_copy(u_hbm.at[sidx(c0 + k + NBUF)], rows_v[k],
                                 gsem[k])

        return carry

    lax.fori_loop(0, NCHUNK // NBUF, quad, 0)
    plsc.subcore_barrier()
    pltpu.sync_copy(acc_sh.at[pl.ds(rs, RPS)], out_hbm.at[c, pl.ds(rs, RPS)])


R = 2560
NBLK = NP // R


def _prep1_body(x_ref, cnt_ref, w_ref, u_ref, dinv_ref):
    i = pl.program_id(0)
    dn = (((0,), (0,)), ((), ()))
    tot = lax.dot_general(cnt_ref[...], jnp.ones((NW, 1), _f32), dn,
                          preferred_element_type=_f32)
    rows = i * R + lax.broadcasted_iota(jnp.int32, (R, 1), 0)
    dinv = jnp.where(rows < N, lax.rsqrt(tot + 1.0), 0.0)
    u_ref[...] = jnp.where(
        rows < N,
        jnp.dot(x_ref[...], w_ref[...], preferred_element_type=_f32) * dinv,
        0.0)
    dinv_ref[...] = dinv


def _prep1(x, ecnt, W1):
    return pl.pallas_call(
        _prep1_body,
        grid=(NBLK,),
        in_specs=[pl.BlockSpec((R, D), lambda i: (i, 0)),
                  pl.BlockSpec((NW, R), lambda i: (0, i)),
                  pl.BlockSpec((D, D), lambda i: (0, 0))],
        out_specs=[pl.BlockSpec((R, D), lambda i: (i, 0)),
                   pl.BlockSpec((R, 1), lambda i: (i, 0))],
        out_shape=[jax.ShapeDtypeStruct((NP, D), _f32),
                   jax.ShapeDtypeStruct((NP, 1), _f32)],
    )(x, ecnt, W1)


def _mid_body(y_ref, u_ref, dinv_ref, b_ref, w_ref, o_ref):
    ys = jnp.sum(y_ref[...], axis=0)
    dinv = dinv_ref[...]
    h = jnp.maximum((ys + u_ref[...]) * dinv + b_ref[...], 0.0)
    o_ref[...] = jnp.dot(h, w_ref[...], preferred_element_type=_f32) * dinv


def _mid(y, u, dinv, b, W):
    return pl.pallas_call(
        _mid_body,
        grid=(NBLK,),
        in_specs=[pl.BlockSpec((NC, R, D), lambda i: (0, i, 0)),
                  pl.BlockSpec((R, D), lambda i: (i, 0)),
                  pl.BlockSpec((R, 1), lambda i: (i, 0)),
                  pl.BlockSpec((1, D), lambda i: (0, 0)),
                  pl.BlockSpec((D, D), lambda i: (0, 0))],
        out_specs=pl.BlockSpec((R, D), lambda i: (i, 0)),
        out_shape=jax.ShapeDtypeStruct((NP, D), _f32),
    )(y, u, dinv, b, W)


def _tail_body(y_ref, u_ref, dinv_ref, b_ref, batch_ref, wc_ref, bc_ref,
               o_ref, sums_ref, cnt_ref):
    i = pl.program_id(0)
    ys = jnp.sum(y_ref[...], axis=0)
    h = jnp.maximum((ys + u_ref[...]) * dinv_ref[...] + b_ref[...], 0.0)
    cols = lax.broadcasted_iota(jnp.int32, (R, G), 1)
    rows = i * R + lax.broadcasted_iota(jnp.int32, (R, 1), 0)
    oh = ((batch_ref[...] == cols) & (rows < N)).astype(_f32)
    dn = (((0,), (0,)), ((), ()))

    @pl.when(i == 0)
    def _():
        sums_ref[...] = jnp.zeros_like(sums_ref)
        cnt_ref[...] = jnp.zeros_like(cnt_ref)

    sums_ref[...] += lax.dot_general(oh, h, dn, preferred_element_type=_f32)
    cnt_ref[...] += lax.dot_general(oh, jnp.ones((R, 1), _f32), dn,
                                    preferred_element_type=_f32)

    @pl.when(i == NBLK - 1)
    def _():
        pooled = sums_ref[...] / jnp.maximum(cnt_ref[...], 1.0)
        o_ref[...] = jnp.dot(pooled, wc_ref[...],
                             preferred_element_type=_f32) + bc_ref[...]


def _tail(y, u, dinv, b, batch_col, Wc, bc):
    return pl.pallas_call(
        _tail_body,
        grid=(NBLK,),
        in_specs=[pl.BlockSpec((NC, R, D), lambda i: (0, i, 0)),
                  pl.BlockSpec((R, D), lambda i: (i, 0)),
                  pl.BlockSpec((R, 1), lambda i: (i, 0)),
                  pl.BlockSpec((1, D), lambda i: (0, 0)),
                  pl.BlockSpec((R, 1), lambda i: (i, 0)),
                  pl.BlockSpec((D, OUT), lambda i: (0, 0)),
                  pl.BlockSpec((1, OUT), lambda i: (0, 0))],
        out_specs=pl.BlockSpec((G, OUT), lambda i: (0, 0)),
        out_shape=jax.ShapeDtypeStruct((G, OUT), _f32),
        scratch_shapes=[pltpu.VMEM((G, D), _f32),
                        pltpu.VMEM((G, 1), _f32)],
    )(y, u, dinv, b, batch_col, Wc, bc)


def kernel(x, edge_index, batch, W1, b1, W2, b2, W3, b3, Wc, bc):
    src = edge_index[0]
    dst = edge_index[1]
    pad_idx = N + (jnp.arange(EP - E, dtype=jnp.int32) % (NP - N))
    src_p = jnp.concatenate([src, pad_idx])
    dst_p = jnp.concatenate([dst, pad_idx])
    src2 = src_p.reshape(NW, EW)
    dst2 = dst_p.reshape(NW, EW)
    dst4 = dst_p.reshape(NW, NCHUNK, 1, CHE)
    zeros1 = jnp.zeros((NP,), _f32)
    zeros2 = jnp.zeros((NP, D), _f32)

    ecnt = _deg(dst2, zeros1)
    u1, dinv = _prep1(x, ecnt, W1)
    y1 = _edge_scatter(u1, src2, dst4, zeros2)
    u2 = _mid(y1, u1, dinv, b1.reshape(1, D), W2)
    y2 = _edge_scatter(u2, src2, dst4, zeros2)
    u3 = _mid(y2, u2, dinv, b2.reshape(1, D), W3)
    y3 = _edge_scatter(u3, src2, dst4, zeros2)
    return _tail(y3, u3, dinv, b3.reshape(1, D), batch.reshape(N, 1),
                 Wc, bc.reshape(1, OUT))
